# Initial kernel scaffold; baseline (speedup 1.0000x reference)
import jax
import jax.numpy as jnp
from jax.experimental import pallas as pl


def kernel(embeddings, operator_idxs, real, imag):
    raise NotImplementedError("write your pallas kernel here")


# speedup vs baseline: 3.3587x; 3.3587x over previous
import jax
import jax.numpy as jnp
from jax import lax
from jax.experimental import pallas as pl
from jax.experimental.pallas import tpu as pltpu
from jax.experimental.pallas import tpu_sc as plsc

BATCH = 16384
DIM = 128
HALF = 64
LANES = 16

_NC = 2
_NS = 16
_NW = _NC * _NS

_ROWS_PER_W = BATCH // _NW
_CHUNK = 128
_NCHUNK = _ROWS_PER_W // _CHUNK


def _sc_body(emb_hbm, idx_hbm, tab_hbm, out_hbm,
             idx_v, emb_v, tab_v, out_v, sem_e, sem_t):
    wid = lax.axis_index("s") * _NC + lax.axis_index("c")
    for chunk in range(_NCHUNK):
        base = wid * _ROWS_PER_W + chunk * _CHUNK
        pltpu.sync_copy(idx_hbm.at[pl.ds(base, _CHUNK)], idx_v)
        emb_cp = pltpu.async_copy(emb_hbm.at[pl.ds(base, _CHUNK)], emb_v, sem_e)
        tab_cp = pltpu.async_copy(tab_hbm.at[idx_v], tab_v, sem_t)
        emb_cp.wait()
        tab_cp.wait()

        def row_body(row, carry):
            for c in range(HALF // LANES):
                lo = c * LANES
                hi = HALF + c * LANES
                er = emb_v[row, pl.ds(lo, LANES)]
                ei = emb_v[row, pl.ds(hi, LANES)]
                rb = tab_v[row, pl.ds(lo, LANES)]
                ib = tab_v[row, pl.ds(hi, LANES)]
                out_v[row, pl.ds(lo, LANES)] = er * rb - ei * ib
                out_v[row, pl.ds(hi, LANES)] = er * ib + ei * rb
            return carry

        lax.fori_loop(0, _CHUNK, row_body, 0)
        pltpu.sync_copy(out_v, out_hbm.at[pl.ds(base, _CHUNK)])


@jax.jit
def _sc_call(embeddings, idx32, table):
    mesh = plsc.VectorSubcoreMesh(core_axis_name="c", subcore_axis_name="s")
    return pl.kernel(
        _sc_body,
        out_type=jax.ShapeDtypeStruct((BATCH, DIM), jnp.float32),
        mesh=mesh,
        scratch_types=[
            pltpu.VMEM((_CHUNK,), jnp.int32),
            pltpu.VMEM((_CHUNK, DIM), jnp.float32),
            pltpu.VMEM((_CHUNK, DIM), jnp.float32),
            pltpu.VMEM((_CHUNK, DIM), jnp.float32),
            pltpu.SemaphoreType.DMA,
            pltpu.SemaphoreType.DMA,
        ],
    )(embeddings, idx32, table)


def kernel(embeddings, operator_idxs, real, imag):
    idx32 = operator_idxs.astype(jnp.int32)
    table = jnp.concatenate([real, imag], axis=-1)
    return _sc_call(embeddings, idx32, table)


# speedup vs baseline: 3.7781x; 1.1249x over previous
import jax
import jax.numpy as jnp
from jax import lax
from jax.experimental import pallas as pl
from jax.experimental.pallas import tpu as pltpu
from jax.experimental.pallas import tpu_sc as plsc

BATCH = 16384
DIM = 128
HALF = 64
LANES = 16

_NC = 2
_NS = 16
_NW = _NC * _NS

_ROWS_PER_W = BATCH // _NW
_CHUNK = 128
_NCHUNK = _ROWS_PER_W // _CHUNK


def _sc_body(emb_hbm, idx_hbm, tab_hbm, out_hbm,
             idx_all, emb0, emb1, tab0, tab1, out0, out1,
             sem_e0, sem_e1, sem_t0, sem_t1, sem_o0, sem_o1):
    emb_v = (emb0, emb1)
    tab_v = (tab0, tab1)
    out_v = (out0, out1)
    sem_e = (sem_e0, sem_e1)
    sem_t = (sem_t0, sem_t1)
    sem_o = (sem_o0, sem_o1)

    wid = lax.axis_index("s") * _NC + lax.axis_index("c")
    rbase = wid * _NCHUNK

    pltpu.sync_copy(idx_hbm.at[pl.ds(rbase, _NCHUNK)], idx_all)

    def start(chunk):
        slot = chunk % 2
        base = (rbase + chunk) * _CHUNK
        e = pltpu.async_copy(emb_hbm.at[pl.ds(base, _CHUNK)], emb_v[slot],
                             sem_e[slot])
        t = pltpu.async_copy(tab_hbm.at[idx_all.at[chunk]], tab_v[slot],
                             sem_t[slot])
        return e, t

    inflight = [None] * _NCHUNK
    out_cp = [None, None]
    inflight[0] = start(0)
    for chunk in range(_NCHUNK):
        slot = chunk % 2
        if chunk + 1 < _NCHUNK:
            inflight[chunk + 1] = start(chunk + 1)
        e, t = inflight[chunk]
        e.wait()
        t.wait()
        if out_cp[slot] is not None:
            out_cp[slot].wait()

        eb, tb, ob = emb_v[slot], tab_v[slot], out_v[slot]

        @plsc.parallel_loop(0, _CHUNK, 1, unroll=2)
        def row_body(row):
            for c in range(HALF // LANES):
                lo = c * LANES
                hi = HALF + c * LANES
                er = eb[row, pl.ds(lo, LANES)]
                ei = eb[row, pl.ds(hi, LANES)]
                rb = tb[row, pl.ds(lo, LANES)]
                ib = tb[row, pl.ds(hi, LANES)]
                ob[row, pl.ds(lo, LANES)] = er * rb - ei * ib
                ob[row, pl.ds(hi, LANES)] = er * ib + ei * rb

        base = (rbase + chunk) * _CHUNK
        out_cp[slot] = pltpu.async_copy(ob, out_hbm.at[pl.ds(base, _CHUNK)],
                                        sem_o[slot])
    for slot in range(2):
        if out_cp[slot] is not None:
            out_cp[slot].wait()


@jax.jit
def _sc_call(embeddings, idx2d, table):
    mesh = plsc.VectorSubcoreMesh(core_axis_name="c", subcore_axis_name="s")
    return pl.kernel(
        _sc_body,
        out_type=jax.ShapeDtypeStruct((BATCH, DIM), jnp.float32),
        mesh=mesh,
        scratch_types=[
            pltpu.VMEM((_NCHUNK, _CHUNK), jnp.int32),
            pltpu.VMEM((_CHUNK, DIM), jnp.float32),
            pltpu.VMEM((_CHUNK, DIM), jnp.float32),
            pltpu.VMEM((_CHUNK, DIM), jnp.float32),
            pltpu.VMEM((_CHUNK, DIM), jnp.float32),
            pltpu.VMEM((_CHUNK, DIM), jnp.float32),
            pltpu.VMEM((_CHUNK, DIM), jnp.float32),
            pltpu.SemaphoreType.DMA,
            pltpu.SemaphoreType.DMA,
            pltpu.SemaphoreType.DMA,
            pltpu.SemaphoreType.DMA,
            pltpu.SemaphoreType.DMA,
            pltpu.SemaphoreType.DMA,
        ],
    )(embeddings, idx2d, table)


def kernel(embeddings, operator_idxs, real, imag):
    idx2d = operator_idxs.astype(jnp.int32).reshape(BATCH // _CHUNK, _CHUNK)
    table = jnp.concatenate([real, imag], axis=-1)
    return _sc_call(embeddings, idx2d, table)


# speedup vs baseline: 3.8305x; 1.0139x over previous
import jax
import jax.numpy as jnp
from jax import lax
from jax.experimental import pallas as pl
from jax.experimental.pallas import tpu as pltpu
from jax.experimental.pallas import tpu_sc as plsc

BATCH = 16384
DIM = 128
HALF = 64
LANES = 16

_NC = 2
_NS = 16
_NW = _NC * _NS

_ROWS_PER_W = BATCH // _NW
_CHUNK = 128
_NCHUNK = _ROWS_PER_W // _CHUNK


def _sc_body(emb_hbm, idx_hbm, real_hbm, imag_hbm, out_hbm,
             idx_all, emb_v, rv_v, iv_v, out_v,
             sem_e, sem_r, sem_i, sem_o):
    wid = lax.axis_index("s") * _NC + lax.axis_index("c")
    rbase = wid * _NCHUNK

    pltpu.sync_copy(idx_hbm.at[pl.ds(rbase, _NCHUNK)], idx_all)

    def start_in(chunk, slot):
        base = (rbase + chunk) * _CHUNK
        pltpu.async_copy(emb_hbm.at[pl.ds(base, _CHUNK)], emb_v.at[slot],
                         sem_e.at[slot])
        pltpu.async_copy(real_hbm.at[idx_all.at[chunk]], rv_v.at[slot],
                         sem_r.at[slot])
        pltpu.async_copy(imag_hbm.at[idx_all.at[chunk]], iv_v.at[slot],
                         sem_i.at[slot])

    def wait_in(slot):
        pltpu.make_async_copy(emb_hbm.at[pl.ds(0, _CHUNK)], emb_v.at[slot],
                              sem_e.at[slot]).wait()
        pltpu.make_async_copy(real_hbm.at[pl.ds(0, _CHUNK)], rv_v.at[slot],
                              sem_r.at[slot]).wait()
        pltpu.make_async_copy(imag_hbm.at[pl.ds(0, _CHUNK)], iv_v.at[slot],
                              sem_i.at[slot]).wait()

    def wait_out(slot):
        pltpu.make_async_copy(out_v.at[slot], out_hbm.at[pl.ds(0, _CHUNK)],
                              sem_o.at[slot]).wait()

    start_in(0, 0)

    def chunk_body(chunk, carry):
        slot = lax.rem(chunk, 2)
        nslot = 1 - slot

        @pl.when(chunk + 1 < _NCHUNK)
        def _():
            start_in(chunk + 1, nslot)

        wait_in(slot)

        @pl.when(chunk >= 2)
        def _():
            wait_out(slot)

        @plsc.parallel_loop(0, _CHUNK, 1, unroll=2)
        def row_body(row):
            for c in range(HALF // LANES):
                lo = c * LANES
                hi = HALF + c * LANES
                er = emb_v[slot, row, pl.ds(lo, LANES)]
                ei = emb_v[slot, row, pl.ds(hi, LANES)]
                rb = rv_v[slot, row, pl.ds(lo, LANES)]
                ib = iv_v[slot, row, pl.ds(lo, LANES)]
                out_v[slot, row, pl.ds(lo, LANES)] = er * rb - ei * ib
                out_v[slot, row, pl.ds(hi, LANES)] = er * ib + ei * rb

        base = (rbase + chunk) * _CHUNK
        pltpu.async_copy(out_v.at[slot], out_hbm.at[pl.ds(base, _CHUNK)],
                         sem_o.at[slot])
        return carry

    lax.fori_loop(0, _NCHUNK, chunk_body, 0)
    wait_out(0)
    wait_out(1)


@jax.jit
def _sc_call(embeddings, idx2d, real, imag):
    mesh = plsc.VectorSubcoreMesh(core_axis_name="c", subcore_axis_name="s")
    return pl.kernel(
        _sc_body,
        out_type=jax.ShapeDtypeStruct((BATCH, DIM), jnp.float32),
        mesh=mesh,
        compiler_params=pltpu.CompilerParams(use_tc_tiling_on_sc=False),
        scratch_types=[
            pltpu.VMEM((_NCHUNK, _CHUNK), jnp.int32),
            pltpu.VMEM((2, _CHUNK, DIM), jnp.float32),
            pltpu.VMEM((2, _CHUNK, HALF), jnp.float32),
            pltpu.VMEM((2, _CHUNK, HALF), jnp.float32),
            pltpu.VMEM((2, _CHUNK, DIM), jnp.float32),
            pltpu.SemaphoreType.DMA((2,)),
            pltpu.SemaphoreType.DMA((2,)),
            pltpu.SemaphoreType.DMA((2,)),
            pltpu.SemaphoreType.DMA((2,)),
        ],
    )(embeddings, idx2d, real, imag)


def kernel(embeddings, operator_idxs, real, imag):
    idx2d = operator_idxs.astype(jnp.int32).reshape(BATCH // _CHUNK, _CHUNK)
    return _sc_call(embeddings, idx2d, real, imag)


# speedup vs baseline: 4.2542x; 1.1106x over previous
import jax
import jax.numpy as jnp
from jax import lax
from jax.experimental import pallas as pl
from jax.experimental.pallas import tpu as pltpu
from jax.experimental.pallas import tpu_sc as plsc

BATCH = 16384
DIM = 128
HALF = 64
LANES = 16
NUM_OPS = 1000

_NC = 2
_NS = 16
_NW = _NC * _NS

_ROWS_PER_W = BATCH // _NW
_CHUNK = 128
_NCHUNK = _ROWS_PER_W // _CHUNK


def _sc_body(emb_hbm, idx_hbm, tab_hbm, out_hbm,
             tab_sh, idx_all, emb_v, tab_v, out_v,
             sem_e, sem_t, sem_o):
    sid = lax.axis_index("s")
    wid = sid * _NC + lax.axis_index("c")
    rbase = wid * _NCHUNK

    @pl.when(sid == 0)
    def _():
        pltpu.sync_copy(tab_hbm, tab_sh)

    pltpu.sync_copy(idx_hbm.at[pl.ds(rbase, _NCHUNK)], idx_all)
    plsc.subcore_barrier()

    def start_in(chunk, slot):
        base = (rbase + chunk) * _CHUNK
        pltpu.async_copy(emb_hbm.at[pl.ds(base, _CHUNK)], emb_v.at[slot],
                         sem_e.at[slot])
        pltpu.async_copy(tab_sh.at[idx_all.at[chunk]], tab_v.at[slot],
                         sem_t.at[slot])

    def wait_in(slot):
        pltpu.make_async_copy(emb_hbm.at[pl.ds(0, _CHUNK)], emb_v.at[slot],
                              sem_e.at[slot]).wait()
        pltpu.make_async_copy(tab_sh.at[pl.ds(0, _CHUNK)], tab_v.at[slot],
                              sem_t.at[slot]).wait()

    def wait_out(slot):
        pltpu.make_async_copy(out_v.at[slot], out_hbm.at[pl.ds(0, _CHUNK)],
                              sem_o.at[slot]).wait()

    start_in(0, 0)

    def chunk_body(chunk, carry):
        slot = lax.rem(chunk, 2)
        nslot = 1 - slot

        @pl.when(chunk + 1 < _NCHUNK)
        def _():
            start_in(chunk + 1, nslot)

        wait_in(slot)

        @pl.when(chunk >= 2)
        def _():
            wait_out(slot)

        @plsc.parallel_loop(0, _CHUNK, 1, unroll=2)
        def row_body(row):
            for c in range(HALF // LANES):
                lo = c * LANES
                hi = HALF + c * LANES
                er = emb_v[slot, row, pl.ds(lo, LANES)]
                ei = emb_v[slot, row, pl.ds(hi, LANES)]
                rb = tab_v[slot, row, pl.ds(lo, LANES)]
                ib = tab_v[slot, row, pl.ds(hi, LANES)]
                out_v[slot, row, pl.ds(lo, LANES)] = er * rb - ei * ib
                out_v[slot, row, pl.ds(hi, LANES)] = er * ib + ei * rb

        base = (rbase + chunk) * _CHUNK
        pltpu.async_copy(out_v.at[slot], out_hbm.at[pl.ds(base, _CHUNK)],
                         sem_o.at[slot])
        return carry

    lax.fori_loop(0, _NCHUNK, chunk_body, 0)
    wait_out(0)
    wait_out(1)


@jax.jit
def _sc_call(embeddings, idx2d, table):
    mesh = plsc.VectorSubcoreMesh(core_axis_name="c", subcore_axis_name="s")
    return pl.kernel(
        _sc_body,
        out_type=jax.ShapeDtypeStruct((BATCH, DIM), jnp.float32),
        mesh=mesh,
        scratch_types=[
            pltpu.VMEM_SHARED((NUM_OPS, DIM), jnp.float32),
            pltpu.VMEM((_NCHUNK, _CHUNK), jnp.int32),
            pltpu.VMEM((2, _CHUNK, DIM), jnp.float32),
            pltpu.VMEM((2, _CHUNK, DIM), jnp.float32),
            pltpu.VMEM((2, _CHUNK, DIM), jnp.float32),
            pltpu.SemaphoreType.DMA((2,)),
            pltpu.SemaphoreType.DMA((2,)),
            pltpu.SemaphoreType.DMA((2,)),
        ],
    )(embeddings, idx2d, table)


def kernel(embeddings, operator_idxs, real, imag):
    idx2d = operator_idxs.astype(jnp.int32).reshape(BATCH // _CHUNK, _CHUNK)
    table = jnp.concatenate([real, imag], axis=-1)
    return _sc_call(embeddings, idx2d, table)


# speedup vs baseline: 4.3027x; 1.0114x over previous
import jax
import jax.numpy as jnp
from jax import lax
from jax.experimental import pallas as pl
from jax.experimental.pallas import tpu as pltpu
from jax.experimental.pallas import tpu_sc as plsc

BATCH = 16384
DIM = 128
HALF = 64
LANES = 16
NUM_OPS = 1000

_NC = 2
_NS = 16
_NW = _NC * _NS

_ROWS_PER_W = BATCH // _NW
_CHUNK = 128
_NCHUNK = _ROWS_PER_W // _CHUNK


def _sc_body(emb_hbm, idx_hbm, tab_hbm, out_hbm,
             tab_sh, idx_all, emb_v, tab_v, out_v,
             sem_e, sem_t, sem_o):
    sid = lax.axis_index("s")
    wid = sid * _NC + lax.axis_index("c")
    rbase = wid * _NCHUNK

    @pl.when(sid == 0)
    def _():
        pltpu.sync_copy(tab_hbm, tab_sh)

    pltpu.sync_copy(idx_hbm.at[pl.ds(rbase, _NCHUNK)], idx_all)

    def start_emb(chunk, slot):
        base = (rbase + chunk) * _CHUNK
        pltpu.async_copy(emb_hbm.at[pl.ds(base, _CHUNK)], emb_v.at[slot],
                         sem_e.at[slot])

    def start_gather(chunk, slot):
        pltpu.async_copy(tab_sh.at[idx_all.at[chunk]], tab_v.at[slot],
                         sem_t.at[slot])

    start_emb(0, 0)
    plsc.subcore_barrier()

    def start_in(chunk, slot):
        start_emb(chunk, slot)
        start_gather(chunk, slot)

    def wait_in(slot):
        pltpu.make_async_copy(emb_hbm.at[pl.ds(0, _CHUNK)], emb_v.at[slot],
                              sem_e.at[slot]).wait()
        pltpu.make_async_copy(tab_sh.at[pl.ds(0, _CHUNK)], tab_v.at[slot],
                              sem_t.at[slot]).wait()

    def wait_out(slot):
        pltpu.make_async_copy(out_v.at[slot], out_hbm.at[pl.ds(0, _CHUNK)],
                              sem_o.at[slot]).wait()

    start_gather(0, 0)

    def chunk_body(chunk, carry):
        slot = lax.rem(chunk, 2)
        nslot = 1 - slot

        @pl.when(chunk + 1 < _NCHUNK)
        def _():
            start_in(chunk + 1, nslot)

        wait_in(slot)

        @pl.when(chunk >= 2)
        def _():
            wait_out(slot)

        @plsc.parallel_loop(0, _CHUNK, 1, unroll=4)
        def row_body(row):
            for c in range(HALF // LANES):
                lo = c * LANES
                hi = HALF + c * LANES
                er = emb_v[slot, row, pl.ds(lo, LANES)]
                ei = emb_v[slot, row, pl.ds(hi, LANES)]
                rb = tab_v[slot, row, pl.ds(lo, LANES)]
                ib = tab_v[slot, row, pl.ds(hi, LANES)]
                out_v[slot, row, pl.ds(lo, LANES)] = er * rb - ei * ib
                out_v[slot, row, pl.ds(hi, LANES)] = er * ib + ei * rb

        base = (rbase + chunk) * _CHUNK
        pltpu.async_copy(out_v.at[slot], out_hbm.at[pl.ds(base, _CHUNK)],
                         sem_o.at[slot])
        return carry

    lax.fori_loop(0, _NCHUNK, chunk_body, 0)
    wait_out(0)
    wait_out(1)


@jax.jit
def _sc_call(embeddings, idx2d, table):
    mesh = plsc.VectorSubcoreMesh(core_axis_name="c", subcore_axis_name="s")
    return pl.kernel(
        _sc_body,
        out_type=jax.ShapeDtypeStruct((BATCH, DIM), jnp.float32),
        mesh=mesh,
        scratch_types=[
            pltpu.VMEM_SHARED((NUM_OPS, DIM), jnp.float32),
            pltpu.VMEM((_NCHUNK, _CHUNK), jnp.int32),
            pltpu.VMEM((2, _CHUNK, DIM), jnp.float32),
            pltpu.VMEM((2, _CHUNK, DIM), jnp.float32),
            pltpu.VMEM((2, _CHUNK, DIM), jnp.float32),
            pltpu.SemaphoreType.DMA((2,)),
            pltpu.SemaphoreType.DMA((2,)),
            pltpu.SemaphoreType.DMA((2,)),
        ],
    )(embeddings, idx2d, table)


def kernel(embeddings, operator_idxs, real, imag):
    idx2d = operator_idxs.astype(jnp.int32).reshape(BATCH // _CHUNK, _CHUNK)
    table = jnp.concatenate([real, imag], axis=-1)
    return _sc_call(embeddings, idx2d, table)
